# baseline (speedup 1.0000x reference)
import functools

import jax
import jax.numpy as jnp
from jax import lax
from jax.experimental import pallas as pl
from jax.experimental.pallas import tpu as pltpu

_N = 5000
_ROWS = 40
_COLS = 128
_PAD = _ROWS * _COLS
_CHUNKS = _ROWS // 8
_IOU_THRESHOLD = 0.5
_BIG = 1 << 30


def _nms_body(x1_ref, y1_ref, x2_ref, y2_ref, s_ref, keep_ref):
    shape = (_ROWS, _COLS)
    lin = (
        lax.broadcasted_iota(jnp.int32, shape, 0) * _COLS
        + lax.broadcasted_iota(jnp.int32, shape, 1)
    )
    valid = lin < _N

    s = s_ref[...]
    neg_inf = jnp.float32(-jnp.inf)
    pos_inf = jnp.float32(jnp.inf)

    s_for_max = jnp.where(valid, s, neg_inf)
    first_max = jnp.max(s_for_max)
    first_min = jnp.min(jnp.where(valid, s, pos_inf))
    sentinel = first_min - 1.0

    mask0 = valid & (s < first_max)
    copy0 = jnp.where(mask0, s, sentinel)
    keep0 = jnp.zeros(shape, dtype=jnp.float32)
    init_count = jnp.sum(mask0.astype(jnp.int32))

    def chunks(a):
        return [a[8 * i:8 * (i + 1), :] for i in range(_CHUNKS)]

    def tree(v):
        def merge(a, b):
            va, ta = a[0], a[1]
            vb, tb = b[0], b[1]
            better = (vb > va) | ((vb == va) & (tb < ta))
            return [jnp.where(better, y, x) for x, y in zip(a, b)]

        cs = [chunks(a) for a in
              (v, lin, x1_ref[...], y1_ref[...], x2_ref[...], y2_ref[...])]
        nodes = [[cs[f][c] for f in range(6)] for c in range(_CHUNKS)]
        m = merge(merge(nodes[0], nodes[1]), merge(nodes[2], nodes[3]))
        m = merge(m, nodes[4])
        for axis, ks in ((0, (4, 2, 1)), (1, (64, 32, 16, 8, 4, 2, 1))):
            for k in ks:
                m = merge(m, [pltpu.roll(f, k, axis) for f in m])
        return m

    t0 = tree(s_for_max)

    def cond_fn(state):
        lag2 = state[-1]
        return (init_count > 0) & (lag2 >= first_min)

    def body_fn(state):
        copy, keep, sv, tv, x1v, y1v, x2v, y2v, lag1, _ = state
        x1 = x1_ref[...]
        y1 = y1_ref[...]
        x2 = x2_ref[...]
        y2 = y2_ref[...]
        area = (x2 - x1) * (y2 - y1)

        si = sv[0:1, :]
        ti = tv[0:1, :]
        x1i = x1v[0:1, :]
        y1i = y1v[0:1, :]
        x2i = x2v[0:1, :]
        y2i = y2v[0:1, :]
        ai = (x2i - x1i) * (y2i - y1i)

        xx = jnp.minimum(x2, x2i) - jnp.maximum(x1, x1i)
        yy = jnp.minimum(y2, y2i) - jnp.maximum(y1, y1i)
        inter = jnp.maximum(xx, 0.0) * jnp.maximum(yy, 0.0)
        iou = inter / ((area + ai) - inter)

        onehot = lin == ti
        alive = si >= first_min
        mask = (copy >= first_min) & (iou <= _IOU_THRESHOLD) & (~onehot)
        keep = jnp.where(onehot & alive, 1.0, keep)
        copy = jnp.where(mask, copy, sentinel)

        m = tree(copy)
        e = m[0][0, 0]
        return (copy, keep, m[0], m[1], m[2], m[3], m[4], m[5], e, lag1)

    state = (copy0, keep0, t0[0], t0[1], t0[2], t0[3], t0[4], t0[5],
             first_max, first_max)
    out = lax.while_loop(cond_fn, body_fn, state)
    keep_ref[...] = out[1]


@functools.partial(jax.jit)
def kernel(boxes, scores):
    pad = _PAD - _N

    def prep(v):
        return jnp.pad(v, (0, pad)).reshape(_ROWS, _COLS)

    x1 = prep(boxes[:, 0])
    y1 = prep(boxes[:, 1])
    x2 = prep(boxes[:, 2])
    y2 = prep(boxes[:, 3])
    s = prep(scores)

    keep = pl.pallas_call(
        _nms_body,
        out_shape=jax.ShapeDtypeStruct((_ROWS, _COLS), jnp.float32),
    )(x1, y1, x2, y2, s)

    m = keep.reshape(_PAD)[:_N]
    return jnp.concatenate([boxes, scores[:, None]], axis=1) * m[:, None]


# speedup vs baseline: 3.1026x; 3.1026x over previous
import functools

import jax
import jax.numpy as jnp
from jax import lax
from jax.experimental import pallas as pl
from jax.experimental.pallas import tpu as pltpu

_N = 5000
_ROWS = 40
_COLS = 128
_PAD = _ROWS * _COLS
_NB = _PAD // _COLS
_THR = 0.5


def _f32(x):
    return x.astype(jnp.float32)


def _recombine(hi_f, lo_f):
    hi = hi_f.astype(jnp.int32)
    lo = lo_f.astype(jnp.int32)
    return lax.bitcast_convert_type((hi << 16) | lo, jnp.float32)


def _nms_v4(C_ref, s_ref, keep_ref, sfr_ref, rank_ref, sC_ref, supp_ref,
            kept_ref):
    f32 = jnp.float32
    shape = (_ROWS, _COLS)
    lin = (
        lax.broadcasted_iota(jnp.int32, shape, 0) * _COLS
        + lax.broadcasted_iota(jnp.int32, shape, 1)
    )
    valid = lin < _N

    s = s_ref[...]
    neg_inf = f32(-jnp.inf)
    pos_inf = f32(jnp.inf)
    s_for_max = jnp.where(valid, s, neg_inf)
    first_max = jnp.max(s_for_max)
    first_min = jnp.min(jnp.where(valid, s, pos_inf))
    init_count = jnp.sum((valid & (s < first_max)).astype(jnp.int32))

    col_iota_i = lax.broadcasted_iota(jnp.int32, (_COLS, 1), 0)
    row_iota_i = lax.broadcasted_iota(jnp.int32, (1, _COLS), 1)
    col_iota_f = _f32(col_iota_i)
    row_iota_f = _f32(row_iota_i)

    sfr_ref[...] = s_for_max
    rank_ref[...] = jnp.zeros(shape, f32)

    def rank_j(jr, _):
        jrow = sfr_ref[pl.ds(jr, 1), :]
        jcol = jnp.swapaxes(jrow, 0, 1)
        jix = col_iota_f + _f32(jr) * 128.0

        def rank_i(ir, _):
            irow = sfr_ref[pl.ds(ir, 1), :]
            iix = row_iota_f + _f32(ir) * 128.0
            cmpv = (jcol > irow) | ((jcol == irow) & (jix < iix))
            cnt = jnp.sum(_f32(cmpv), axis=0, keepdims=True)
            rank_ref[pl.ds(ir, 1), :] += cnt
            return 0

        lax.fori_loop(0, _NB, rank_i, 0)
        return 0

    lax.fori_loop(0, _NB, rank_j, 0)

    def perm_k(kb, _):
        kcol = col_iota_f + _f32(kb) * 128.0

        def perm_acc(ir, acc):
            rrow = rank_ref[pl.ds(ir, 1), :]
            pt = _f32(rrow == kcol)
            cb = C_ref[pl.ds(ir * _COLS, _COLS), :]
            return acc + jnp.dot(pt, cb, preferred_element_type=f32)

        blk = lax.fori_loop(0, _NB, perm_acc, jnp.zeros((_COLS, 16), f32))
        sC_ref[pl.ds(kb * _COLS, _COLS), :] = blk
        return 0

    lax.fori_loop(0, _NB, perm_k, 0)

    kept_ref[...] = jnp.zeros(shape, f32)

    def init_supp(bb, _):
        blk_t = jnp.swapaxes(sC_ref[pl.ds(bb * _COLS, _COLS), :], 0, 1)
        ss = _recombine(blk_t[8:9, :], blk_t[9:10, :])
        klin = row_iota_i + bb * _COLS
        supp = (klin >= _N) | ((ss == first_max) & (klin > 0))
        supp_ref[pl.ds(bb, 1), :] = _f32(supp)
        return 0

    lax.fori_loop(0, _NB, init_supp, 0)

    def block_cols(blk):
        x1 = _recombine(blk[:, 0:1], blk[:, 1:2])
        y1 = _recombine(blk[:, 2:3], blk[:, 3:4])
        x2 = _recombine(blk[:, 4:5], blk[:, 5:6])
        y2 = _recombine(blk[:, 6:7], blk[:, 7:8])
        return x1, y1, x2, y2, (x2 - x1) * (y2 - y1)

    def block_rows(blk_t):
        x1 = _recombine(blk_t[0:1, :], blk_t[1:2, :])
        y1 = _recombine(blk_t[2:3, :], blk_t[3:4, :])
        x2 = _recombine(blk_t[4:5, :], blk_t[5:6, :])
        y2 = _recombine(blk_t[6:7, :], blk_t[7:8, :])
        return x1, y1, x2, y2, (x2 - x1) * (y2 - y1)

    def iou_gt(cols, rows):
        x1c, y1c, x2c, y2c, ac = cols
        x1r, y1r, x2r, y2r, ar = rows
        xx = jnp.minimum(x2c, x2r) - jnp.maximum(x1c, x1r)
        yy = jnp.minimum(y2c, y2r) - jnp.maximum(y1c, y1r)
        inter = jnp.maximum(xx, 0.0) * jnp.maximum(yy, 0.0)
        iou = inter / ((ac + ar) - inter)
        return iou > _THR

    def scan_b(bb, _):
        blk = sC_ref[pl.ds(bb * _COLS, _COLS), :]
        blk_t = jnp.swapaxes(blk, 0, 1)
        cols = block_cols(blk)
        rows = block_rows(blk_t)
        m = _f32(iou_gt(cols, rows) & (col_iota_f < row_iota_f))

        dead0 = supp_ref[pl.ds(bb, 1), :]
        kept0 = jnp.zeros((1, _COLS), f32)
        rem0 = jnp.sum((1.0 - dead0))

        def round_cond(st):
            return st[2] > 0.5

        def round_body(st):
            dead, kept, _ = st
            deadc = jnp.swapaxes(dead, 0, 1)
            keptc = jnp.swapaxes(kept, 0, 1)
            threat = jnp.max(m * (1.0 - deadc), axis=0, keepdims=True)
            kill = jnp.max(m * keptc, axis=0, keepdims=True)
            unknown = (1.0 - dead) * (1.0 - kept)
            newkept = unknown * (1.0 - threat)
            newdead = unknown * kill
            kept = kept + newkept
            dead = dead + newdead
            rem = jnp.sum((1.0 - dead) * (1.0 - kept))
            return dead, kept, rem

        _, kept, _ = lax.while_loop(round_cond, round_body,
                                    (dead0, kept0, rem0))
        kept_ref[pl.ds(bb, 1), :] = kept
        keptc = jnp.swapaxes(kept, 0, 1)

        def sweep_c(cc, _):
            cblk = sC_ref[pl.ds(cc * _COLS, _COLS), :]
            crows = block_rows(jnp.swapaxes(cblk, 0, 1))
            kill2 = jnp.max(_f32(iou_gt(cols, crows)) * keptc,
                            axis=0, keepdims=True)
            supp_ref[pl.ds(cc, 1), :] = jnp.maximum(
                supp_ref[pl.ds(cc, 1), :], kill2)
            return 0

        lax.fori_loop(bb + 1, _NB, sweep_c, 0)
        return 0

    lax.fori_loop(0, _NB, scan_b, 0)

    gate = _f32(init_count > 0)

    def unperm_i(ir, _):
        rrow = rank_ref[pl.ds(ir, 1), :]

        def unperm_acc(kb, acc):
            kcol = col_iota_f + _f32(kb) * 128.0
            pt = _f32(rrow == kcol)
            ks = kept_ref[pl.ds(kb, 1), :]
            return acc + jnp.dot(ks, pt, preferred_element_type=f32)

        row = lax.fori_loop(0, _NB, unperm_acc, jnp.zeros((1, _COLS), f32))
        keep_ref[pl.ds(ir, 1), :] = row * gate
        return 0

    lax.fori_loop(0, _NB, unperm_i, 0)


def _split16(x):
    bits = lax.bitcast_convert_type(x, jnp.int32)
    hi = ((bits >> 16) & 0xFFFF).astype(jnp.float32)
    lo = (bits & 0xFFFF).astype(jnp.float32)
    return hi, lo


@functools.partial(jax.jit)
def kernel(boxes, scores):
    pad = _PAD - _N

    def prep(v):
        return jnp.pad(v, (0, pad)).reshape(_ROWS, _COLS)

    s = prep(scores)

    cols = []
    for arr in (boxes[:, 0], boxes[:, 1], boxes[:, 2], boxes[:, 3], scores):
        hi, lo = _split16(jnp.pad(arr, (0, pad)))
        cols.extend([hi, lo])
    cols.extend([jnp.zeros(_PAD, jnp.float32)] * 6)
    C = jnp.stack(cols, axis=1)

    f32 = jnp.float32
    keep = pl.pallas_call(
        _nms_v4,
        out_shape=jax.ShapeDtypeStruct((_ROWS, _COLS), f32),
        scratch_shapes=[
            pltpu.VMEM((_ROWS, _COLS), f32),
            pltpu.VMEM((_ROWS, _COLS), f32),
            pltpu.VMEM((_PAD, 16), f32),
            pltpu.VMEM((_ROWS, _COLS), f32),
            pltpu.VMEM((_ROWS, _COLS), f32),
        ],
    )(C, s)

    m = keep.reshape(_PAD)[:_N]
    return jnp.concatenate([boxes, scores[:, None]], axis=1) * m[:, None]
